# baseline (speedup 1.0000x reference)
import jax
import jax.numpy as jnp
from jax import lax
from jax.experimental import pallas as pl
from jax.experimental.pallas import tpu as pltpu
from jax.experimental.pallas import tpu_sc as plsc

IM_SIZE = 1024
RADIUS = 128
BATCH = 512

_NW = 32
_RPW = BATCH // _NW
_FLAT = _RPW * IM_SIZE
_STRIDE = IM_SIZE + 8
_NPAD = IM_SIZE + RADIUS
_NEG = -3.0e30
_PADV = -1000.0 - float(IM_SIZE)


def _body(hf_hbm, out_hbm, rows_v, out_v, p_v, sem):
    wid = lax.axis_index("s") * 2 + lax.axis_index("c")
    base = wid * _FLAT
    copies = [
        pltpu.async_copy(
            hf_hbm.at[pl.ds(base + r * IM_SIZE, IM_SIZE)],
            rows_v.at[pl.ds(r * _STRIDE, IM_SIZE)],
            sem,
        )
        for r in range(_RPW)
    ]
    for c in copies:
        c.wait()

    lane_base = lax.iota(jnp.int32, 16) * _STRIDE
    negv = jnp.full((16,), _NEG, jnp.float32)
    padv = jnp.full((16,), _PADV, jnp.float32)
    onei = jnp.full((16,), 1, jnp.int32)
    onef = jnp.full((16,), 1.0, jnp.float32)

    def pass_a(j_in, carries):
        idxs, jfs, pms = carries
        new_idx, new_jf, new_pm = [], [], []
        for k in range(7):
            j = (k + 1) * RADIUS + j_in
            v = plsc.load_gather(rows_v, [idxs[k]])
            g = v - jfs[k]
            pm = jnp.maximum(g, pms[k])
            p_v[j] = pm
            new_idx.append(idxs[k] + onei)
            new_jf.append(jfs[k] + onef)
            new_pm.append(pm)
        return tuple(new_idx), tuple(new_jf), tuple(new_pm)

    init_a = (
        tuple(lane_base + (k + 1) * RADIUS for k in range(7)),
        tuple(jnp.full((16,), float((k + 1) * RADIUS), jnp.float32) for k in range(7)),
        (negv,) * 7,
    )
    plsc.parallel_loop(0, RADIUS, 1, unroll=8, carry=init_a)(
        lambda i, c: pass_a(i, c)
    )

    def pass_b(jj, carries):
        j_in = RADIUS - 1 - jj
        idxs, jfs, sms = carries
        new_idx, new_jf, new_sm = [], [], []
        for b in range(8):
            j = b * RADIUS + j_in
            v = plsc.load_gather(rows_v, [idxs[b]])
            g = v - jfs[b]
            pn = p_v[j + RADIUS] if b < 7 else padv
            o = jnp.maximum(jnp.maximum(sms[b], pn) - g, 0.0)
            plsc.store_scatter(out_v, [idxs[b]], o)
            new_idx.append(idxs[b] - onei)
            new_jf.append(jfs[b] - onef)
            new_sm.append(jnp.maximum(sms[b], g))
        return tuple(new_idx), tuple(new_jf), tuple(new_sm)

    init_b = (
        tuple(lane_base + (b * RADIUS + RADIUS - 1) for b in range(8)),
        tuple(
            jnp.full((16,), float(b * RADIUS + RADIUS - 1), jnp.float32)
            for b in range(8)
        ),
        (negv,) * 8,
    )
    plsc.parallel_loop(0, RADIUS, 1, unroll=8, carry=init_b)(
        lambda i, c: pass_b(i, c)
    )

    copies = [
        pltpu.async_copy(
            out_v.at[pl.ds(r * _STRIDE, IM_SIZE)],
            out_hbm.at[pl.ds(base + r * IM_SIZE, IM_SIZE)],
            sem,
        )
        for r in range(_RPW)
    ]
    for c in copies:
        c.wait()


def kernel(height_field):
    mesh = plsc.VectorSubcoreMesh(core_axis_name="c", subcore_axis_name="s")
    f = pl.kernel(
        _body,
        out_type=jax.ShapeDtypeStruct((BATCH * IM_SIZE,), jnp.float32),
        mesh=mesh,
        scratch_types=[
            pltpu.VMEM((_RPW * _STRIDE,), jnp.float32),
            pltpu.VMEM((_RPW * _STRIDE,), jnp.float32),
            pltpu.VMEM((_NPAD, 16), jnp.float32),
            pltpu.SemaphoreType.DMA,
        ],
        compiler_params=pltpu.CompilerParams(
            use_tc_tiling_on_sc=False, needs_layout_passes=False
        ),
    )
    return f(height_field.reshape(-1)).reshape(BATCH, IM_SIZE)


# speedup vs baseline: 1.0097x; 1.0097x over previous
import jax
import jax.numpy as jnp
from jax import lax
from jax.experimental import pallas as pl
from jax.experimental.pallas import tpu as pltpu
from jax.experimental.pallas import tpu_sc as plsc

IM_SIZE = 1024
RADIUS = 128
BATCH = 512

_NW = 32
_RPW = BATCH // _NW
_FLAT = _RPW * IM_SIZE
_STRIDE = IM_SIZE + 8
_NPAD = IM_SIZE + RADIUS
_NEG = -3.0e30
_PADV = -1000.0 - float(IM_SIZE)


def _body(hf_hbm, out_hbm, rows_v, out_v, p_v, sem):
    wid = lax.axis_index("s") * 2 + lax.axis_index("c")
    base = wid * _FLAT
    copies = [
        pltpu.async_copy(
            hf_hbm.at[pl.ds(base + r * IM_SIZE, IM_SIZE)],
            rows_v.at[pl.ds(r * _STRIDE, IM_SIZE)],
            sem,
        )
        for r in range(_RPW)
    ]
    for c in copies:
        c.wait()

    lane_base = lax.iota(jnp.int32, 16) * _STRIDE
    negv = jnp.full((16,), _NEG, jnp.float32)
    padv = jnp.full((16,), _PADV, jnp.float32)
    onei = jnp.full((16,), 1, jnp.int32)
    onef = jnp.full((16,), 1.0, jnp.float32)

    def pass_a(j_in, carries):
        idxs, jfs, pms = carries
        new_idx, new_jf, new_pm = [], [], []
        for k in range(7):
            j = (k + 1) * RADIUS + j_in
            v = plsc.load_gather(rows_v, [idxs[k]])
            g = v - jfs[k]
            pm = jnp.maximum(g, pms[k])
            p_v[j] = pm
            new_idx.append(idxs[k] + onei)
            new_jf.append(jfs[k] + onef)
            new_pm.append(pm)
        return tuple(new_idx), tuple(new_jf), tuple(new_pm)

    init_a = (
        tuple(lane_base + (k + 1) * RADIUS for k in range(7)),
        tuple(jnp.full((16,), float((k + 1) * RADIUS), jnp.float32) for k in range(7)),
        (negv,) * 7,
    )
    plsc.parallel_loop(0, RADIUS, 1, unroll=4, carry=init_a)(
        lambda i, c: pass_a(i, c)
    )

    def pass_b(jj, carries):
        j_in = RADIUS - 1 - jj
        idxs, jfs, sms = carries
        new_idx, new_jf, new_sm = [], [], []
        for b in range(8):
            j = b * RADIUS + j_in
            v = plsc.load_gather(rows_v, [idxs[b]])
            g = v - jfs[b]
            pn = p_v[j + RADIUS] if b < 7 else padv
            o = jnp.maximum(jnp.maximum(sms[b], pn) - g, 0.0)
            plsc.store_scatter(out_v, [idxs[b]], o)
            new_idx.append(idxs[b] - onei)
            new_jf.append(jfs[b] - onef)
            new_sm.append(jnp.maximum(sms[b], g))
        return tuple(new_idx), tuple(new_jf), tuple(new_sm)

    init_b = (
        tuple(lane_base + (b * RADIUS + RADIUS - 1) for b in range(8)),
        tuple(
            jnp.full((16,), float(b * RADIUS + RADIUS - 1), jnp.float32)
            for b in range(8)
        ),
        (negv,) * 8,
    )
    plsc.parallel_loop(0, RADIUS, 1, unroll=4, carry=init_b)(
        lambda i, c: pass_b(i, c)
    )

    copies = [
        pltpu.async_copy(
            out_v.at[pl.ds(r * _STRIDE, IM_SIZE)],
            out_hbm.at[pl.ds(base + r * IM_SIZE, IM_SIZE)],
            sem,
        )
        for r in range(_RPW)
    ]
    for c in copies:
        c.wait()


def kernel(height_field):
    mesh = plsc.VectorSubcoreMesh(core_axis_name="c", subcore_axis_name="s")
    f = pl.kernel(
        _body,
        out_type=jax.ShapeDtypeStruct((BATCH * IM_SIZE,), jnp.float32),
        mesh=mesh,
        scratch_types=[
            pltpu.VMEM((_RPW * _STRIDE,), jnp.float32),
            pltpu.VMEM((_RPW * _STRIDE,), jnp.float32),
            pltpu.VMEM((_NPAD, 16), jnp.float32),
            pltpu.SemaphoreType.DMA,
        ],
        compiler_params=pltpu.CompilerParams(
            use_tc_tiling_on_sc=False, needs_layout_passes=False
        ),
    )
    return f(height_field.reshape(-1)).reshape(BATCH, IM_SIZE)
